# baseline (speedup 1.0000x reference)
import functools

import jax
import jax.numpy as jnp
from jax import lax
from jax.experimental import pallas as pl
from jax.experimental.pallas import tpu as pltpu
from jax.experimental.pallas import tpu_sc as plsc

T = 16384
D = 4096
E = 64
K = 8

NCH = 1
TCH = T // NCH
RB = 256

NW = 32
RPW = TCH // NW
UNROLL = 16


def _noisy_body(x_ref, w_ref, b_ref, eps_ref, out_ref):
    acc = jnp.dot(x_ref[...], w_ref[...], preferred_element_type=jnp.float32)
    acc = acc + b_ref[...]
    out_ref[...] = acc[:, :E] + eps_ref[...] * jax.nn.softplus(acc[:, E:])


def _tc_noisy(x, Wc, bc, eps):
    return pl.pallas_call(
        _noisy_body,
        grid=(TCH // RB,),
        in_specs=[
            pl.BlockSpec((RB, D), lambda i: (i, 0)),
            pl.BlockSpec((D, 2 * E), lambda i: (0, 0)),
            pl.BlockSpec((1, 2 * E), lambda i: (0, 0)),
            pl.BlockSpec((RB, E), lambda i: (i, 0)),
        ],
        out_specs=pl.BlockSpec((RB, E), lambda i: (i, 0)),
        out_shape=jax.ShapeDtypeStruct((TCH, E), jnp.float32),
    )(x, Wc, bc, eps)


def _merge16(ka, va, kb, vb):
    rk = lax.rev(kb, (0,))
    rv = lax.rev(vb, (0,))
    take_b = jnp.logical_or(rk > ka, jnp.logical_and(rk == ka, rv < va))
    mk = jnp.where(take_b, rk, ka)
    mv = jnp.where(take_b, rv, va)
    return plsc.sort_key_val(mk, mv, descending=True)


@functools.partial(
    pl.kernel,
    mesh=plsc.VectorSubcoreMesh(core_axis_name="c", subcore_axis_name="s"),
    compiler_params=pltpu.CompilerParams(needs_layout_passes=False),
    out_type=[
        jax.ShapeDtypeStruct((TCH * E,), jnp.float32),
        jax.ShapeDtypeStruct((TCH * K,), jnp.int32),
    ],
    scratch_types=[
        pltpu.VMEM((RPW * E,), jnp.float32),
        pltpu.VMEM((RPW * E,), jnp.float32),
        pltpu.VMEM((RPW * K,), jnp.int32),
    ],
)
def _sc_router(noisy_hbm, probs_hbm, idx_hbm, noisy_v, probs_v, idx_v):
    wid = lax.axis_index("s") * 2 + lax.axis_index("c")
    base = wid * RPW
    pltpu.sync_copy(noisy_hbm.at[pl.ds(base * E, RPW * E)], noisy_v)

    lanes = lax.iota(jnp.int32, 16)
    lane8 = lanes < 8
    lanemod8 = jnp.bitwise_and(lanes, 7)
    zeros16 = jnp.zeros((16,), jnp.float32)
    idx_consts = [lanes + c * 16 for c in range(E // 16)]

    def do_row(r):
        sorted_chunks = [
            plsc.sort_key_val(noisy_v[pl.ds(r * E + c * 16, 16)],
                              idx_consts[c], descending=True)
            for c in range(E // 16)
        ]
        k01, v01 = _merge16(*sorted_chunks[0], *sorted_chunks[1])
        k23, v23 = _merge16(*sorted_chunks[2], *sorted_chunks[3])
        tk, tv = _merge16(k01, v01, k23, v23)

        m0 = jnp.max(tk)
        u = jnp.where(lane8, jnp.exp(tk - m0), 0.0)
        p = u / jnp.sum(u)

        for c in range(E // 16):
            probs_v[pl.ds(r * E + c * 16, 16)] = zeros16
        plsc.store_scatter(probs_v, [r * E + tv], p, mask=lane8)
        plsc.store_scatter(idx_v, [r * K + lanemod8], tv, mask=lane8)

    @plsc.parallel_loop(0, RPW, 1, unroll=UNROLL)
    def _rows(r):
        do_row(r)

    pltpu.sync_copy(probs_v, probs_hbm.at[pl.ds(base * E, RPW * E)])
    pltpu.sync_copy(idx_v, idx_hbm.at[pl.ds(base * K, RPW * K)])


@jax.jit
def kernel(x, W1, b1, W2, b2):
    eps = jax.random.normal(jax.random.key(42), (T, E), dtype=jnp.float32)
    Wc = jnp.concatenate([W1, W2], axis=1)
    bc = jnp.concatenate([b1, b2]).reshape(1, 2 * E)

    probs_chunks = []
    idx_chunks = []
    for c in range(NCH):
        sl = slice(c * TCH, (c + 1) * TCH)
        noisy = _tc_noisy(x[sl], Wc, bc, eps[sl])
        pf, ix = _sc_router(noisy.reshape(TCH * E))
        probs_chunks.append(pf.reshape(TCH, E))
        idx_chunks.append(ix.reshape(TCH, K))

    return (jnp.concatenate(probs_chunks, axis=0),
            jnp.concatenate(idx_chunks, axis=0))


# speedup vs baseline: 1.0895x; 1.0895x over previous
import functools

import jax
import jax.numpy as jnp
from jax import lax
from jax.experimental import pallas as pl
from jax.experimental.pallas import tpu as pltpu
from jax.experimental.pallas import tpu_sc as plsc

T = 16384
D = 4096
E = 64
K = 8

NCH = 1
TCH = T // NCH
RB = 512

NW = 32
RPW = TCH // NW
UNROLL = 16


def _noisy_body(x_ref, w_ref, b_ref, eps_ref, out_ref):
    acc = jnp.dot(x_ref[...], w_ref[...], preferred_element_type=jnp.float32)
    acc = acc + b_ref[...]
    out_ref[...] = acc[:, :E] + eps_ref[...] * jax.nn.softplus(acc[:, E:])


def _tc_noisy(x, Wc, bc, eps):
    return pl.pallas_call(
        _noisy_body,
        grid=(TCH // RB,),
        in_specs=[
            pl.BlockSpec((RB, D), lambda i: (i, 0)),
            pl.BlockSpec((D, 2 * E), lambda i: (0, 0)),
            pl.BlockSpec((1, 2 * E), lambda i: (0, 0)),
            pl.BlockSpec((RB, E), lambda i: (i, 0)),
        ],
        out_specs=pl.BlockSpec((RB, E), lambda i: (i, 0)),
        out_shape=jax.ShapeDtypeStruct((TCH, E), jnp.float32),
    )(x, Wc, bc, eps)


def _merge16(ka, va, kb, vb):
    rk = lax.rev(kb, (0,))
    rv = lax.rev(vb, (0,))
    take_b = jnp.logical_or(rk > ka, jnp.logical_and(rk == ka, rv < va))
    mk = jnp.where(take_b, rk, ka)
    mv = jnp.where(take_b, rv, va)
    return plsc.sort_key_val(mk, mv, descending=True)


@functools.partial(
    pl.kernel,
    mesh=plsc.VectorSubcoreMesh(core_axis_name="c", subcore_axis_name="s"),
    compiler_params=pltpu.CompilerParams(needs_layout_passes=False),
    out_type=[
        jax.ShapeDtypeStruct((TCH * E,), jnp.float32),
        jax.ShapeDtypeStruct((TCH * K,), jnp.int32),
    ],
    scratch_types=[
        pltpu.VMEM((RPW * E,), jnp.float32),
        pltpu.VMEM((RPW * E,), jnp.float32),
        pltpu.VMEM((RPW * K,), jnp.int32),
    ],
)
def _sc_router(noisy_hbm, probs_hbm, idx_hbm, noisy_v, probs_v, idx_v):
    wid = lax.axis_index("s") * 2 + lax.axis_index("c")
    base = wid * RPW
    pltpu.sync_copy(noisy_hbm.at[pl.ds(base * E, RPW * E)], noisy_v)

    lanes = lax.iota(jnp.int32, 16)
    lane8 = lanes < 8
    lanemod8 = jnp.bitwise_and(lanes, 7)
    zeros16 = jnp.zeros((16,), jnp.float32)
    idx_consts = [lanes + c * 16 for c in range(E // 16)]

    def do_row(r):
        sorted_chunks = [
            plsc.sort_key_val(noisy_v[pl.ds(r * E + c * 16, 16)],
                              idx_consts[c], descending=True)
            for c in range(E // 16)
        ]
        k01, v01 = _merge16(*sorted_chunks[0], *sorted_chunks[1])
        k23, v23 = _merge16(*sorted_chunks[2], *sorted_chunks[3])
        tk, tv = _merge16(k01, v01, k23, v23)

        m0 = jnp.max(tk)
        u = jnp.where(lane8, jnp.exp(tk - m0), 0.0)
        p = u / jnp.sum(u)

        for c in range(E // 16):
            probs_v[pl.ds(r * E + c * 16, 16)] = zeros16
        plsc.store_scatter(probs_v, [r * E + tv], p, mask=lane8)
        plsc.store_scatter(idx_v, [r * K + lanemod8], tv, mask=lane8)

    @plsc.parallel_loop(0, RPW, 1, unroll=UNROLL)
    def _rows(r):
        do_row(r)

    pltpu.sync_copy(probs_v, probs_hbm.at[pl.ds(base * E, RPW * E)])
    pltpu.sync_copy(idx_v, idx_hbm.at[pl.ds(base * K, RPW * K)])


@jax.jit
def kernel(x, W1, b1, W2, b2):
    eps = jax.random.normal(jax.random.key(42), (T, E), dtype=jnp.float32)
    Wc = jnp.concatenate([W1, W2], axis=1)
    bc = jnp.concatenate([b1, b2]).reshape(1, 2 * E)

    probs_chunks = []
    idx_chunks = []
    for c in range(NCH):
        sl = slice(c * TCH, (c + 1) * TCH)
        noisy = _tc_noisy(x[sl], Wc, bc, eps[sl])
        pf, ix = _sc_router(noisy.reshape(TCH * E))
        probs_chunks.append(pf.reshape(TCH, E))
        idx_chunks.append(ix.reshape(TCH, K))

    return (jnp.concatenate(probs_chunks, axis=0),
            jnp.concatenate(idx_chunks, axis=0))


# speedup vs baseline: 1.1313x; 1.0383x over previous
import functools

import jax
import jax.numpy as jnp
from jax import lax
from jax.experimental import pallas as pl
from jax.experimental.pallas import tpu as pltpu
from jax.experimental.pallas import tpu_sc as plsc

T = 16384
D = 4096
E = 64
K = 8

NCH = 1
TCH = T // NCH
RB = 512

NW = 32
RPW = TCH // NW
UNROLL = 8


def _noisy_body(x_ref, w_ref, b_ref, eps_ref, out_ref):
    acc = jnp.dot(x_ref[...], w_ref[...], preferred_element_type=jnp.float32)
    acc = acc + b_ref[...]
    out_ref[...] = acc[:, :E] + eps_ref[...] * jax.nn.softplus(acc[:, E:])


def _tc_noisy(x, Wc, bc, eps):
    return pl.pallas_call(
        _noisy_body,
        grid=(TCH // RB,),
        in_specs=[
            pl.BlockSpec((RB, D), lambda i: (i, 0)),
            pl.BlockSpec((D, 2 * E), lambda i: (0, 0)),
            pl.BlockSpec((1, 2 * E), lambda i: (0, 0)),
            pl.BlockSpec((RB, E), lambda i: (i, 0)),
        ],
        out_specs=pl.BlockSpec((RB, E), lambda i: (i, 0)),
        out_shape=jax.ShapeDtypeStruct((TCH, E), jnp.float32),
    )(x, Wc, bc, eps)


def _merge16(ka, va, kb, vb):
    rk = lax.rev(kb, (0,))
    rv = lax.rev(vb, (0,))
    take_b = jnp.logical_or(rk > ka, jnp.logical_and(rk == ka, rv < va))
    mk = jnp.where(take_b, rk, ka)
    mv = jnp.where(take_b, rv, va)
    return plsc.sort_key_val(mk, mv, descending=True)


@functools.partial(
    pl.kernel,
    mesh=plsc.VectorSubcoreMesh(core_axis_name="c", subcore_axis_name="s"),
    compiler_params=pltpu.CompilerParams(needs_layout_passes=False),
    out_type=[
        jax.ShapeDtypeStruct((TCH * E,), jnp.float32),
        jax.ShapeDtypeStruct((TCH * K,), jnp.int32),
    ],
    scratch_types=[
        pltpu.VMEM((RPW * E,), jnp.float32),
        pltpu.VMEM((RPW * E,), jnp.float32),
        pltpu.VMEM((RPW * K,), jnp.int32),
    ],
)
def _sc_router(noisy_hbm, probs_hbm, idx_hbm, noisy_v, probs_v, idx_v):
    wid = lax.axis_index("s") * 2 + lax.axis_index("c")
    base = wid * RPW
    pltpu.sync_copy(noisy_hbm.at[pl.ds(base * E, RPW * E)], noisy_v)

    lanes = lax.iota(jnp.int32, 16)
    lane8 = lanes < 8
    lanemod8 = jnp.bitwise_and(lanes, 7)
    zeros16 = jnp.zeros((16,), jnp.float32)
    idx_consts = [lanes + c * 16 for c in range(E // 16)]

    def do_row(r):
        sorted_chunks = [
            plsc.sort_key_val(noisy_v[pl.ds(r * E + c * 16, 16)],
                              idx_consts[c], descending=True)
            for c in range(E // 16)
        ]
        k01, v01 = _merge16(*sorted_chunks[0], *sorted_chunks[1])
        k23, v23 = _merge16(*sorted_chunks[2], *sorted_chunks[3])
        tk, tv = _merge16(k01, v01, k23, v23)

        u = jnp.where(lane8, jnp.exp(tk), 0.0)
        p = u / jnp.sum(u)

        plsc.store_scatter(probs_v, [r * E + tv], p, mask=lane8)
        plsc.store_scatter(idx_v, [r * K + lanemod8], tv, mask=lane8)

    @plsc.parallel_loop(0, RPW * E // 16, 1, unroll=32)
    def _zero(j):
        probs_v[pl.ds(j * 16, 16)] = zeros16

    @plsc.parallel_loop(0, RPW, 1, unroll=UNROLL)
    def _rows(r):
        do_row(r)

    pltpu.sync_copy(probs_v, probs_hbm.at[pl.ds(base * E, RPW * E)])
    pltpu.sync_copy(idx_v, idx_hbm.at[pl.ds(base * K, RPW * K)])


@jax.jit
def kernel(x, W1, b1, W2, b2):
    eps = jax.random.normal(jax.random.key(42), (T, E), dtype=jnp.float32)
    Wc = jnp.concatenate([W1, W2], axis=1)
    bc = jnp.concatenate([b1, b2]).reshape(1, 2 * E)

    probs_chunks = []
    idx_chunks = []
    for c in range(NCH):
        sl = slice(c * TCH, (c + 1) * TCH)
        noisy = _tc_noisy(x[sl], Wc, bc, eps[sl])
        pf, ix = _sc_router(noisy.reshape(TCH * E))
        probs_chunks.append(pf.reshape(TCH, E))
        idx_chunks.append(ix.reshape(TCH, K))

    return (jnp.concatenate(probs_chunks, axis=0),
            jnp.concatenate(idx_chunks, axis=0))
